# baseline (speedup 1.0000x reference)
import functools

import jax
import jax.numpy as jnp
from jax.experimental import pallas as pl
from jax.experimental.pallas import tpu as pltpu

_TB = 128
_HB = 512


def _router_body(x_ref, gate_ref,
                 p1_ref, p2_ref, w1_ref, w2_ref, gid_ref, valid_ref,
                 *, E, T, NB):
    logits = jnp.dot(x_ref[...], gate_ref[...],
                     preferred_element_type=jnp.float32)
    ids8 = jax.lax.broadcasted_iota(jnp.int32, (T, E), 1)
    a1 = jnp.argmax(logits, axis=1, keepdims=True)
    s1 = jnp.max(logits, axis=1, keepdims=True)
    masked = jnp.where(ids8 == a1, -jnp.inf, logits)
    a2 = jnp.argmax(masked, axis=1, keepdims=True)
    s2 = jnp.max(masked, axis=1, keepdims=True)
    e2 = jnp.exp(s2 - s1)
    w1_ref[...] = 1.0 / (1.0 + e2)
    w2_ref[...] = e2 / (1.0 + e2)

    oh1 = (ids8 == a1)
    oh2 = (ids8 == a2)
    oh1f = oh1.astype(jnp.float32)
    oh2f = oh2.astype(jnp.float32)

    tri = (jax.lax.broadcasted_iota(jnp.int32, (T, T), 0)
           > jax.lax.broadcasted_iota(jnp.int32, (T, T), 1)
           ).astype(jnp.bfloat16)
    s1cnt = jnp.dot(tri, oh1.astype(jnp.bfloat16),
                    preferred_element_type=jnp.float32)
    s2cnt = jnp.dot(tri, oh2.astype(jnp.bfloat16),
                    preferred_element_type=jnp.float32)
    c0 = jnp.sum(oh1f, axis=0, keepdims=True)
    c1 = jnp.sum(oh2f, axis=0, keepdims=True)
    c = c0 + c1
    pc = jnp.ceil(c / _TB) * _TB
    triu8 = (jax.lax.broadcasted_iota(jnp.int32, (E, E), 0)
             < jax.lax.broadcasted_iota(jnp.int32, (E, E), 1)
             ).astype(jnp.float32)
    offs = jnp.dot(pc, triu8, preferred_element_type=jnp.float32)

    rank1 = jnp.sum(oh1f * s1cnt, axis=1, keepdims=True)
    rank2 = jnp.sum(oh2f * (s2cnt + c0), axis=1, keepdims=True)
    off1 = jnp.sum(oh1f * offs, axis=1, keepdims=True)
    off2 = jnp.sum(oh2f * offs, axis=1, keepdims=True)
    p1_ref[...] = (off1 + rank1).astype(jnp.int32)
    p2_ref[...] = (off2 + rank2).astype(jnp.int32)

    sb = (jax.lax.broadcasted_iota(jnp.int32, (64, 1), 0)
          .astype(jnp.float32) * _TB)
    gid = jnp.sum((offs <= sb).astype(jnp.float32), axis=1,
                  keepdims=True) - 1.0
    total = jnp.sum(pc, axis=1, keepdims=True)
    gid_ref[...] = jnp.clip(gid, 0.0, E - 1.0).astype(jnp.int32)
    valid_ref[...] = (sb < total).astype(jnp.int32)


def _expert_body(gid_sref, valid_sref, xb_ref, p1_ref, p2_ref,
                 wk_ref, bk_ref, wv_ref, bv_ref, y_ref, *, T, D, H):
    b = pl.program_id(0)

    @pl.when(valid_sref[b] == 1)
    def _compute():
        slot = (jax.lax.broadcasted_iota(jnp.int32, (_TB, T), 0)
                + b * _TB)
        g = ((p1_ref[...] == slot) | (p2_ref[...] == slot)
             ).astype(jnp.bfloat16)
        xs = jnp.dot(g, xb_ref[...],
                     preferred_element_type=jnp.float32).astype(jnp.bfloat16)
        acc = jnp.zeros((_TB, D), dtype=jnp.float32)
        for c in range(H // _HB):
            sl = slice(c * _HB, (c + 1) * _HB)
            h = jax.nn.gelu(
                jnp.dot(xs, wk_ref[0][:, sl].astype(jnp.bfloat16),
                        preferred_element_type=jnp.float32)
                + bk_ref[0, :, sl])
            acc = acc + jnp.dot(h.astype(jnp.bfloat16),
                                wv_ref[0][sl, :].astype(jnp.bfloat16),
                                preferred_element_type=jnp.float32)
        y_ref[...] = acc + bv_ref[0]

    @pl.when(valid_sref[b] == 0)
    def _pad():
        y_ref[...] = jnp.zeros_like(y_ref)


def _combine_body(xb_ref, p1_ref, p2_ref, w1_ref, w2_ref, y_ref,
                  wks_ref, bks_ref, wvs_ref, bvs_ref, out_ref, *, NS):
    hb = pl.program_id(1)

    @pl.when(hb == 0)
    def _combine():
        s_ids = jax.lax.broadcasted_iota(jnp.int32, (out_ref.shape[0], NS), 1)
        cb = (jnp.where(p1_ref[...] == s_ids, w1_ref[...], 0.0)
              + jnp.where(p2_ref[...] == s_ids, w2_ref[...], 0.0))
        out_ref[...] = (jnp.dot(cb, y_ref[...],
                                preferred_element_type=jnp.float32)
                        + bvs_ref[...])

    hs = jax.nn.gelu(jnp.dot(xb_ref[...],
                             wks_ref[...].astype(jnp.bfloat16),
                             preferred_element_type=jnp.float32)
                     + bks_ref[...])
    out_ref[...] += jnp.dot(hs.astype(jnp.bfloat16),
                            wvs_ref[...].astype(jnp.bfloat16),
                            preferred_element_type=jnp.float32)


@functools.partial(jax.jit, static_argnames=())
def kernel(x, gate_kernel, Wk, bk, Wv, bv, Wk_s, bk_s, Wv_s, bv_s):
    B, S, D = x.shape
    T = B * S
    E = gate_kernel.shape[1]
    H = Wk.shape[2]
    NB = (2 * T) // _TB + E
    NS = NB * _TB
    NHB = H // _HB

    x2 = x.reshape(T, D)
    xb = x2.astype(jnp.bfloat16)
    bk3 = bk.reshape(E, 1, H)
    bv3 = bv.reshape(E, 1, D)
    bks2 = bk_s.reshape(1, H)
    bvs2 = bv_s.reshape(1, D)

    router = pl.pallas_call(
        functools.partial(_router_body, E=E, T=T, NB=NB),
        in_specs=[pl.BlockSpec((T, D), lambda: (0, 0)),
                  pl.BlockSpec((D, E), lambda: (0, 0))],
        out_specs=[pl.BlockSpec((T, 1), lambda: (0, 0)),
                   pl.BlockSpec((T, 1), lambda: (0, 0)),
                   pl.BlockSpec((T, 1), lambda: (0, 0)),
                   pl.BlockSpec((T, 1), lambda: (0, 0)),
                   pl.BlockSpec((64, 1), lambda: (0, 0)),
                   pl.BlockSpec((64, 1), lambda: (0, 0))],
        out_shape=[jax.ShapeDtypeStruct((T, 1), jnp.int32),
                   jax.ShapeDtypeStruct((T, 1), jnp.int32),
                   jax.ShapeDtypeStruct((T, 1), jnp.float32),
                   jax.ShapeDtypeStruct((T, 1), jnp.float32),
                   jax.ShapeDtypeStruct((64, 1), jnp.int32),
                   jax.ShapeDtypeStruct((64, 1), jnp.int32)],
    )(x2, gate_kernel)
    p1c, p2c, w1c, w2c, gid64, valid64 = router
    p1r = p1c.reshape(1, T)
    p2r = p2c.reshape(1, T)
    gids = gid64.reshape(64)[:NB]
    valid = valid64.reshape(64)[:NB]

    y = pl.pallas_call(
        functools.partial(_expert_body, T=T, D=D, H=H),
        grid_spec=pltpu.PrefetchScalarGridSpec(
            num_scalar_prefetch=2,
            grid=(NB,),
            in_specs=[
                pl.BlockSpec((T, D), lambda b, g, v: (0, 0)),
                pl.BlockSpec((1, T), lambda b, g, v: (0, 0)),
                pl.BlockSpec((1, T), lambda b, g, v: (0, 0)),
                pl.BlockSpec((1, D, H), lambda b, g, v: (g[b], 0, 0)),
                pl.BlockSpec((1, 1, H), lambda b, g, v: (g[b], 0, 0)),
                pl.BlockSpec((1, H, D), lambda b, g, v: (g[b], 0, 0)),
                pl.BlockSpec((1, 1, D), lambda b, g, v: (g[b], 0, 0)),
            ],
            out_specs=pl.BlockSpec((_TB, D), lambda b, g, v: (b, 0)),
        ),
        out_shape=jax.ShapeDtypeStruct((NS, D), jnp.float32),
        compiler_params=pltpu.CompilerParams(
            dimension_semantics=("arbitrary",)),
    )(gids, valid, xb, p1r, p2r, Wk, bk3, Wv, bv3)

    TBC = 256
    out = pl.pallas_call(
        functools.partial(_combine_body, NS=NS),
        grid=(T // TBC, NHB),
        in_specs=[
            pl.BlockSpec((TBC, D), lambda t, h: (t, 0)),
            pl.BlockSpec((TBC, 1), lambda t, h: (t, 0)),
            pl.BlockSpec((TBC, 1), lambda t, h: (t, 0)),
            pl.BlockSpec((TBC, 1), lambda t, h: (t, 0)),
            pl.BlockSpec((TBC, 1), lambda t, h: (t, 0)),
            pl.BlockSpec((NS, D), lambda t, h: (0, 0)),
            pl.BlockSpec((D, _HB), lambda t, h: (0, h)),
            pl.BlockSpec((1, _HB), lambda t, h: (0, h)),
            pl.BlockSpec((_HB, D), lambda t, h: (h, 0)),
            pl.BlockSpec((1, D), lambda t, h: (0, 0)),
        ],
        out_specs=pl.BlockSpec((TBC, D), lambda t, h: (t, 0)),
        out_shape=jax.ShapeDtypeStruct((T, D), jnp.float32),
        compiler_params=pltpu.CompilerParams(
            dimension_semantics=("arbitrary", "arbitrary")),
    )(xb, p1c, p2c, w1c, w2c, y, Wk_s, bks2, Wv_s, bvs2)

    return (out.reshape(B, S, D), jnp.float32(0.0))


# speedup vs baseline: 1.1001x; 1.1001x over previous
import functools

import jax
import jax.numpy as jnp
from jax.experimental import pallas as pl
from jax.experimental.pallas import tpu as pltpu

_TB = 256
_HB = 512


def _router_body(x_ref, gate_ref,
                 p1_ref, p2_ref, w1_ref, w2_ref, gid_ref, valid_ref,
                 *, E, T, NB):
    logits = jnp.dot(x_ref[...], gate_ref[...],
                     preferred_element_type=jnp.float32)
    ids8 = jax.lax.broadcasted_iota(jnp.int32, (T, E), 1)
    a1 = jnp.argmax(logits, axis=1, keepdims=True)
    s1 = jnp.max(logits, axis=1, keepdims=True)
    masked = jnp.where(ids8 == a1, -jnp.inf, logits)
    a2 = jnp.argmax(masked, axis=1, keepdims=True)
    s2 = jnp.max(masked, axis=1, keepdims=True)
    e2 = jnp.exp(s2 - s1)
    w1_ref[...] = 1.0 / (1.0 + e2)
    w2_ref[...] = e2 / (1.0 + e2)

    oh1 = (ids8 == a1)
    oh2 = (ids8 == a2)
    oh1f = oh1.astype(jnp.float32)
    oh2f = oh2.astype(jnp.float32)

    tri = (jax.lax.broadcasted_iota(jnp.int32, (T, T), 0)
           > jax.lax.broadcasted_iota(jnp.int32, (T, T), 1)
           ).astype(jnp.bfloat16)
    s1cnt = jnp.dot(tri, oh1.astype(jnp.bfloat16),
                    preferred_element_type=jnp.float32)
    s2cnt = jnp.dot(tri, oh2.astype(jnp.bfloat16),
                    preferred_element_type=jnp.float32)
    c0 = jnp.sum(oh1f, axis=0, keepdims=True)
    c1 = jnp.sum(oh2f, axis=0, keepdims=True)
    c = c0 + c1
    pc = jnp.ceil(c / _TB) * _TB
    triu8 = (jax.lax.broadcasted_iota(jnp.int32, (E, E), 0)
             < jax.lax.broadcasted_iota(jnp.int32, (E, E), 1)
             ).astype(jnp.float32)
    offs = jnp.dot(pc, triu8, preferred_element_type=jnp.float32)

    rank1 = jnp.sum(oh1f * s1cnt, axis=1, keepdims=True)
    rank2 = jnp.sum(oh2f * (s2cnt + c0), axis=1, keepdims=True)
    off1 = jnp.sum(oh1f * offs, axis=1, keepdims=True)
    off2 = jnp.sum(oh2f * offs, axis=1, keepdims=True)
    p1_ref[...] = (off1 + rank1).astype(jnp.int32)
    p2_ref[...] = (off2 + rank2).astype(jnp.int32)

    sb = (jax.lax.broadcasted_iota(jnp.int32, (64, 1), 0)
          .astype(jnp.float32) * _TB)
    gid = jnp.sum((offs <= sb).astype(jnp.float32), axis=1,
                  keepdims=True) - 1.0
    total = jnp.sum(pc, axis=1, keepdims=True)
    gid_ref[...] = jnp.clip(gid, 0.0, E - 1.0).astype(jnp.int32)
    valid_ref[...] = (sb < total).astype(jnp.int32)


def _expert_body(gid_sref, valid_sref, xb_ref, p1_ref, p2_ref,
                 wk_ref, bk_ref, wv_ref, bv_ref, y_ref, *, T, D, H):
    b = pl.program_id(0)

    @pl.when(valid_sref[b] == 1)
    def _compute():
        slot = (jax.lax.broadcasted_iota(jnp.int32, (_TB, T), 0)
                + b * _TB)
        g = ((p1_ref[...] == slot) | (p2_ref[...] == slot)
             ).astype(jnp.bfloat16)
        xs = jnp.dot(g, xb_ref[...],
                     preferred_element_type=jnp.float32).astype(jnp.bfloat16)
        acc = jnp.zeros((_TB, D), dtype=jnp.float32)
        for c in range(H // _HB):
            sl = slice(c * _HB, (c + 1) * _HB)
            h = jax.nn.gelu(
                jnp.dot(xs, wk_ref[0][:, sl].astype(jnp.bfloat16),
                        preferred_element_type=jnp.float32)
                + bk_ref[0, :, sl])
            acc = acc + jnp.dot(h.astype(jnp.bfloat16),
                                wv_ref[0][sl, :].astype(jnp.bfloat16),
                                preferred_element_type=jnp.float32)
        y_ref[...] = acc + bv_ref[0]

    @pl.when(valid_sref[b] == 0)
    def _pad():
        y_ref[...] = jnp.zeros_like(y_ref)


def _combine_body(xb_ref, p1_ref, p2_ref, w1_ref, w2_ref, y_ref,
                  wks_ref, bks_ref, wvs_ref, bvs_ref, out_ref, *, NS):
    hb = pl.program_id(1)

    @pl.when(hb == 0)
    def _combine():
        s_ids = jax.lax.broadcasted_iota(jnp.int32, (out_ref.shape[0], NS), 1)
        cb = (jnp.where(p1_ref[...] == s_ids, w1_ref[...], 0.0)
              + jnp.where(p2_ref[...] == s_ids, w2_ref[...], 0.0))
        out_ref[...] = (jnp.dot(cb, y_ref[...],
                                preferred_element_type=jnp.float32)
                        + bvs_ref[...])

    hs = jax.nn.gelu(jnp.dot(xb_ref[...],
                             wks_ref[...].astype(jnp.bfloat16),
                             preferred_element_type=jnp.float32)
                     + bks_ref[...])
    out_ref[...] += jnp.dot(hs.astype(jnp.bfloat16),
                            wvs_ref[...].astype(jnp.bfloat16),
                            preferred_element_type=jnp.float32)


@functools.partial(jax.jit, static_argnames=())
def kernel(x, gate_kernel, Wk, bk, Wv, bv, Wk_s, bk_s, Wv_s, bv_s):
    B, S, D = x.shape
    T = B * S
    E = gate_kernel.shape[1]
    H = Wk.shape[2]
    NB = (2 * T) // _TB + E
    NS = NB * _TB
    NHB = H // _HB

    x2 = x.reshape(T, D)
    xb = x2.astype(jnp.bfloat16)
    bk3 = bk.reshape(E, 1, H)
    bv3 = bv.reshape(E, 1, D)
    bks2 = bk_s.reshape(1, H)
    bvs2 = bv_s.reshape(1, D)

    router = pl.pallas_call(
        functools.partial(_router_body, E=E, T=T, NB=NB),
        in_specs=[pl.BlockSpec((T, D), lambda: (0, 0)),
                  pl.BlockSpec((D, E), lambda: (0, 0))],
        out_specs=[pl.BlockSpec((T, 1), lambda: (0, 0)),
                   pl.BlockSpec((T, 1), lambda: (0, 0)),
                   pl.BlockSpec((T, 1), lambda: (0, 0)),
                   pl.BlockSpec((T, 1), lambda: (0, 0)),
                   pl.BlockSpec((64, 1), lambda: (0, 0)),
                   pl.BlockSpec((64, 1), lambda: (0, 0))],
        out_shape=[jax.ShapeDtypeStruct((T, 1), jnp.int32),
                   jax.ShapeDtypeStruct((T, 1), jnp.int32),
                   jax.ShapeDtypeStruct((T, 1), jnp.float32),
                   jax.ShapeDtypeStruct((T, 1), jnp.float32),
                   jax.ShapeDtypeStruct((64, 1), jnp.int32),
                   jax.ShapeDtypeStruct((64, 1), jnp.int32)],
    )(x2, gate_kernel)
    p1c, p2c, w1c, w2c, gid64, valid64 = router
    p1r = p1c.reshape(1, T)
    p2r = p2c.reshape(1, T)
    gids = gid64.reshape(64)[:NB]
    valid = valid64.reshape(64)[:NB]

    y = pl.pallas_call(
        functools.partial(_expert_body, T=T, D=D, H=H),
        grid_spec=pltpu.PrefetchScalarGridSpec(
            num_scalar_prefetch=2,
            grid=(NB,),
            in_specs=[
                pl.BlockSpec((T, D), lambda b, g, v: (0, 0)),
                pl.BlockSpec((1, T), lambda b, g, v: (0, 0)),
                pl.BlockSpec((1, T), lambda b, g, v: (0, 0)),
                pl.BlockSpec((1, D, H), lambda b, g, v: (g[b], 0, 0)),
                pl.BlockSpec((1, 1, H), lambda b, g, v: (g[b], 0, 0)),
                pl.BlockSpec((1, H, D), lambda b, g, v: (g[b], 0, 0)),
                pl.BlockSpec((1, 1, D), lambda b, g, v: (g[b], 0, 0)),
            ],
            out_specs=pl.BlockSpec((_TB, D), lambda b, g, v: (b, 0)),
        ),
        out_shape=jax.ShapeDtypeStruct((NS, D), jnp.float32),
        compiler_params=pltpu.CompilerParams(
            dimension_semantics=("arbitrary",)),
    )(gids, valid, xb, p1r, p2r, Wk, bk3, Wv, bv3)

    TBC = 256
    out = pl.pallas_call(
        functools.partial(_combine_body, NS=NS),
        grid=(T // TBC, NHB),
        in_specs=[
            pl.BlockSpec((TBC, D), lambda t, h: (t, 0)),
            pl.BlockSpec((TBC, 1), lambda t, h: (t, 0)),
            pl.BlockSpec((TBC, 1), lambda t, h: (t, 0)),
            pl.BlockSpec((TBC, 1), lambda t, h: (t, 0)),
            pl.BlockSpec((TBC, 1), lambda t, h: (t, 0)),
            pl.BlockSpec((NS, D), lambda t, h: (0, 0)),
            pl.BlockSpec((D, _HB), lambda t, h: (0, h)),
            pl.BlockSpec((1, _HB), lambda t, h: (0, h)),
            pl.BlockSpec((_HB, D), lambda t, h: (h, 0)),
            pl.BlockSpec((1, D), lambda t, h: (0, 0)),
        ],
        out_specs=pl.BlockSpec((TBC, D), lambda t, h: (t, 0)),
        out_shape=jax.ShapeDtypeStruct((T, D), jnp.float32),
        compiler_params=pltpu.CompilerParams(
            dimension_semantics=("arbitrary", "arbitrary")),
    )(xb, p1c, p2c, w1c, w2c, y, Wk_s, bks2, Wv_s, bvs2)

    return (out.reshape(B, S, D), jnp.float32(0.0))


# speedup vs baseline: 1.1395x; 1.0358x over previous
import functools

import jax
import jax.numpy as jnp
from jax.experimental import pallas as pl
from jax.experimental.pallas import tpu as pltpu

_TB = 256
_HB = 512


def _router_body(x_ref, gate_ref,
                 p1_ref, p2_ref, w1_ref, w2_ref, gid_ref, valid_ref,
                 first_ref, issue_ref, ngid_ref, par_ref,
                 *, E, T, NB):
    logits = jnp.dot(x_ref[...], gate_ref[...],
                     preferred_element_type=jnp.float32)
    ids8 = jax.lax.broadcasted_iota(jnp.int32, (T, E), 1)
    a1 = jnp.argmax(logits, axis=1, keepdims=True)
    s1 = jnp.max(logits, axis=1, keepdims=True)
    masked = jnp.where(ids8 == a1, -jnp.inf, logits)
    a2 = jnp.argmax(masked, axis=1, keepdims=True)
    s2 = jnp.max(masked, axis=1, keepdims=True)
    e2 = jnp.exp(s2 - s1)
    w1_ref[...] = 1.0 / (1.0 + e2)
    w2_ref[...] = e2 / (1.0 + e2)

    oh1 = (ids8 == a1)
    oh2 = (ids8 == a2)
    oh1f = oh1.astype(jnp.float32)
    oh2f = oh2.astype(jnp.float32)

    tri = (jax.lax.broadcasted_iota(jnp.int32, (T, T), 0)
           > jax.lax.broadcasted_iota(jnp.int32, (T, T), 1)
           ).astype(jnp.bfloat16)
    s1cnt = jnp.dot(tri, oh1.astype(jnp.bfloat16),
                    preferred_element_type=jnp.float32)
    s2cnt = jnp.dot(tri, oh2.astype(jnp.bfloat16),
                    preferred_element_type=jnp.float32)
    c0 = jnp.sum(oh1f, axis=0, keepdims=True)
    c1 = jnp.sum(oh2f, axis=0, keepdims=True)
    c = c0 + c1
    pc = jnp.ceil(c / _TB) * _TB
    triu8 = (jax.lax.broadcasted_iota(jnp.int32, (E, E), 0)
             < jax.lax.broadcasted_iota(jnp.int32, (E, E), 1)
             ).astype(jnp.float32)
    offs = jnp.dot(pc, triu8, preferred_element_type=jnp.float32)

    rank1 = jnp.sum(oh1f * s1cnt, axis=1, keepdims=True)
    rank2 = jnp.sum(oh2f * (s2cnt + c0), axis=1, keepdims=True)
    off1 = jnp.sum(oh1f * offs, axis=1, keepdims=True)
    off2 = jnp.sum(oh2f * offs, axis=1, keepdims=True)
    p1_ref[...] = (off1 + rank1).astype(jnp.int32)
    p2_ref[...] = (off2 + rank2).astype(jnp.int32)

    sb = (jax.lax.broadcasted_iota(jnp.int32, (64, 1), 0)
          .astype(jnp.float32) * _TB)
    gid = jnp.sum((offs <= sb).astype(jnp.float32), axis=1,
                  keepdims=True) - 1.0
    total = jnp.sum(pc, axis=1, keepdims=True)
    gidc = jnp.clip(gid, 0.0, E - 1.0)
    gid_ref[...] = gidc.astype(jnp.int32)
    validb = sb < total
    valid_ref[...] = validb.astype(jnp.int32)

    ids8b = jax.lax.broadcasted_iota(jnp.int32, (64, E), 1).astype(jnp.float32)
    ohg = (gidc == ids8b).astype(jnp.float32)
    offg = jnp.sum(ohg * offs, axis=1, keepdims=True)
    pcg = jnp.sum(ohg * pc, axis=1, keepdims=True)
    used = (c > 0.0).astype(jnp.float32)
    tri_incl = (jax.lax.broadcasted_iota(jnp.int32, (E, E), 0)
                <= jax.lax.broadcasted_iota(jnp.int32, (E, E), 1)
                ).astype(jnp.float32)
    cumu = jnp.dot(used, tri_incl,
                   preferred_element_type=jnp.float32)
    dist = jnp.sum(ohg * cumu, axis=1, keepdims=True) - 1.0
    par_ref[...] = (dist - 2.0 * jnp.floor(dist / 2.0)).astype(jnp.int32)
    firstb = (sb == offg) & validb
    first_ref[...] = firstb.astype(jnp.int32)
    end_sb = offg + pcg
    issue_ref[...] = (firstb & (end_sb < total)).astype(jnp.int32)
    ngid = jnp.sum((offs <= end_sb).astype(jnp.float32), axis=1,
                   keepdims=True) - 1.0
    ngid_ref[...] = jnp.clip(ngid, 0.0, E - 1.0).astype(jnp.int32)


def _expert_body(gid_sref, valid_sref, first_sref, issue_sref, ngid_sref,
                 par_sref, xb_ref, p1_ref, p2_ref,
                 wk_hbm, bk_ref, wv_hbm, bv_ref, y_ref,
                 wk_buf, wv_buf, sk0, sk1, sv0, sv1, *, T, D, H):
    b = pl.program_id(0)
    sks = (sk0, sk1)
    svs = (sv0, sv1)

    def _start(e_idx, slot):
        pltpu.make_async_copy(wk_hbm.at[e_idx], wk_buf.at[slot],
                              sks[slot]).start()
        pltpu.make_async_copy(wv_hbm.at[e_idx], wv_buf.at[slot],
                              svs[slot]).start()

    def _wait(slot):
        pltpu.make_async_copy(wk_hbm.at[0], wk_buf.at[slot],
                              sks[slot]).wait()
        pltpu.make_async_copy(wv_hbm.at[0], wv_buf.at[slot],
                              svs[slot]).wait()

    par_b = par_sref[b]

    @pl.when(b == 0)
    def _kickoff():
        _start(gid_sref[0], 0)

    for slot in (0, 1):
        @pl.when((issue_sref[b] == 1) & (par_b == 1 - slot))
        def _issue(slot=slot):
            _start(ngid_sref[b], slot)

    for slot in (0, 1):
        @pl.when((first_sref[b] == 1) & (par_b == slot))
        def _sync(slot=slot):
            _wait(slot)

    @pl.when(valid_sref[b] == 1)
    def _compute():
        slot = (jax.lax.broadcasted_iota(jnp.int32, (_TB, T), 0)
                + b * _TB)
        g = ((p1_ref[...] == slot) | (p2_ref[...] == slot)
             ).astype(jnp.bfloat16)
        xs = jnp.dot(g, xb_ref[...],
                     preferred_element_type=jnp.float32).astype(jnp.bfloat16)
        acc = jnp.zeros((_TB, D), dtype=jnp.float32)
        for c in range(H // _HB):
            sl = slice(c * _HB, (c + 1) * _HB)
            h = jax.nn.gelu(
                jnp.dot(xs, wk_buf[par_b, :, sl].astype(jnp.bfloat16),
                        preferred_element_type=jnp.float32)
                + bk_ref[0, :, sl])
            acc = acc + jnp.dot(h.astype(jnp.bfloat16),
                                wv_buf[par_b, sl, :].astype(jnp.bfloat16),
                                preferred_element_type=jnp.float32)
        y_ref[...] = acc + bv_ref[0]

    @pl.when(valid_sref[b] == 0)
    def _pad():
        y_ref[...] = jnp.zeros_like(y_ref)


def _combine_body(xb_ref, p1_ref, p2_ref, w1_ref, w2_ref, y_ref,
                  wks_ref, bks_ref, wvs_ref, bvs_ref, out_ref, *, NS):
    hb = pl.program_id(1)

    @pl.when(hb == 0)
    def _combine():
        s_ids = jax.lax.broadcasted_iota(jnp.int32, (out_ref.shape[0], NS), 1)
        cb = (jnp.where(p1_ref[...] == s_ids, w1_ref[...], 0.0)
              + jnp.where(p2_ref[...] == s_ids, w2_ref[...], 0.0))
        out_ref[...] = (jnp.dot(cb, y_ref[...],
                                preferred_element_type=jnp.float32)
                        + bvs_ref[...])

    hs = jax.nn.gelu(jnp.dot(xb_ref[...],
                             wks_ref[...].astype(jnp.bfloat16),
                             preferred_element_type=jnp.float32)
                     + bks_ref[...])
    out_ref[...] += jnp.dot(hs.astype(jnp.bfloat16),
                            wvs_ref[...].astype(jnp.bfloat16),
                            preferred_element_type=jnp.float32)


@functools.partial(jax.jit, static_argnames=())
def kernel(x, gate_kernel, Wk, bk, Wv, bv, Wk_s, bk_s, Wv_s, bv_s):
    B, S, D = x.shape
    T = B * S
    E = gate_kernel.shape[1]
    H = Wk.shape[2]
    NB = (2 * T) // _TB + E
    NS = NB * _TB
    NHB = H // _HB

    x2 = x.reshape(T, D)
    xb = x2.astype(jnp.bfloat16)
    bk3 = bk.reshape(E, 1, H)
    bv3 = bv.reshape(E, 1, D)
    bks2 = bk_s.reshape(1, H)
    bvs2 = bv_s.reshape(1, D)

    router = pl.pallas_call(
        functools.partial(_router_body, E=E, T=T, NB=NB),
        in_specs=[pl.BlockSpec((T, D), lambda: (0, 0)),
                  pl.BlockSpec((D, E), lambda: (0, 0))],
        out_specs=[pl.BlockSpec((T, 1), lambda: (0, 0)),
                   pl.BlockSpec((T, 1), lambda: (0, 0)),
                   pl.BlockSpec((T, 1), lambda: (0, 0)),
                   pl.BlockSpec((T, 1), lambda: (0, 0)),
                   pl.BlockSpec((64, 1), lambda: (0, 0)),
                   pl.BlockSpec((64, 1), lambda: (0, 0)),
                   pl.BlockSpec((64, 1), lambda: (0, 0)),
                   pl.BlockSpec((64, 1), lambda: (0, 0)),
                   pl.BlockSpec((64, 1), lambda: (0, 0)),
                   pl.BlockSpec((64, 1), lambda: (0, 0))],
        out_shape=[jax.ShapeDtypeStruct((T, 1), jnp.int32),
                   jax.ShapeDtypeStruct((T, 1), jnp.int32),
                   jax.ShapeDtypeStruct((T, 1), jnp.float32),
                   jax.ShapeDtypeStruct((T, 1), jnp.float32),
                   jax.ShapeDtypeStruct((64, 1), jnp.int32),
                   jax.ShapeDtypeStruct((64, 1), jnp.int32),
                   jax.ShapeDtypeStruct((64, 1), jnp.int32),
                   jax.ShapeDtypeStruct((64, 1), jnp.int32),
                   jax.ShapeDtypeStruct((64, 1), jnp.int32),
                   jax.ShapeDtypeStruct((64, 1), jnp.int32)],
    )(x2, gate_kernel)
    (p1c, p2c, w1c, w2c, gid64, valid64,
     first64, issue64, ngid64, par64) = router
    p1r = p1c.reshape(1, T)
    p2r = p2c.reshape(1, T)
    gids = gid64.reshape(64)[:NB]
    valid = valid64.reshape(64)[:NB]
    first = first64.reshape(64)[:NB]
    issue = issue64.reshape(64)[:NB]
    ngid = ngid64.reshape(64)[:NB]
    par = par64.reshape(64)[:NB]

    y = pl.pallas_call(
        functools.partial(_expert_body, T=T, D=D, H=H),
        grid_spec=pltpu.PrefetchScalarGridSpec(
            num_scalar_prefetch=6,
            grid=(NB,),
            in_specs=[
                pl.BlockSpec((T, D), lambda b, *s: (0, 0)),
                pl.BlockSpec((1, T), lambda b, *s: (0, 0)),
                pl.BlockSpec((1, T), lambda b, *s: (0, 0)),
                pl.BlockSpec(memory_space=pl.ANY),
                pl.BlockSpec((1, 1, H), lambda b, *s: (s[0][b], 0, 0)),
                pl.BlockSpec(memory_space=pl.ANY),
                pl.BlockSpec((1, 1, D), lambda b, *s: (s[0][b], 0, 0)),
            ],
            out_specs=pl.BlockSpec((_TB, D), lambda b, *s: (b, 0)),
            scratch_shapes=[
                pltpu.VMEM((2, D, H), jnp.float32),
                pltpu.VMEM((2, H, D), jnp.float32),
                pltpu.SemaphoreType.DMA,
                pltpu.SemaphoreType.DMA,
                pltpu.SemaphoreType.DMA,
                pltpu.SemaphoreType.DMA,
            ],
        ),
        out_shape=jax.ShapeDtypeStruct((NS, D), jnp.float32),
        compiler_params=pltpu.CompilerParams(
            dimension_semantics=("arbitrary",)),
    )(gids, valid, first, issue, ngid, par, xb, p1r, p2r, Wk, bk3, Wv, bv3)

    TBC = 256
    out = pl.pallas_call(
        functools.partial(_combine_body, NS=NS),
        grid=(T // TBC, NHB),
        in_specs=[
            pl.BlockSpec((TBC, D), lambda t, h: (t, 0)),
            pl.BlockSpec((TBC, 1), lambda t, h: (t, 0)),
            pl.BlockSpec((TBC, 1), lambda t, h: (t, 0)),
            pl.BlockSpec((TBC, 1), lambda t, h: (t, 0)),
            pl.BlockSpec((TBC, 1), lambda t, h: (t, 0)),
            pl.BlockSpec((NS, D), lambda t, h: (0, 0)),
            pl.BlockSpec((D, _HB), lambda t, h: (0, h)),
            pl.BlockSpec((1, _HB), lambda t, h: (0, h)),
            pl.BlockSpec((_HB, D), lambda t, h: (h, 0)),
            pl.BlockSpec((1, D), lambda t, h: (0, 0)),
        ],
        out_specs=pl.BlockSpec((TBC, D), lambda t, h: (t, 0)),
        out_shape=jax.ShapeDtypeStruct((T, D), jnp.float32),
        compiler_params=pltpu.CompilerParams(
            dimension_semantics=("arbitrary", "arbitrary")),
    )(xb, p1c, p2c, w1c, w2c, y, Wk_s, bks2, Wv_s, bvs2)

    return (out.reshape(B, S, D), jnp.float32(0.0))


# speedup vs baseline: 1.1560x; 1.0144x over previous
import functools

import jax
import jax.numpy as jnp
from jax.experimental import pallas as pl
from jax.experimental.pallas import tpu as pltpu

_TB = 256
_HB = 512


def _router_body(x_ref, gate_ref,
                 p1_ref, p2_ref, w1_ref, w2_ref, gid_ref, valid_ref,
                 first_ref, issue_ref, ngid_ref, par_ref,
                 *, E, T, NB):
    logits = jnp.dot(x_ref[...], gate_ref[...],
                     preferred_element_type=jnp.float32)
    ids8 = jax.lax.broadcasted_iota(jnp.int32, (T, E), 1)
    a1 = jnp.argmax(logits, axis=1, keepdims=True)
    s1 = jnp.max(logits, axis=1, keepdims=True)
    masked = jnp.where(ids8 == a1, -jnp.inf, logits)
    a2 = jnp.argmax(masked, axis=1, keepdims=True)
    s2 = jnp.max(masked, axis=1, keepdims=True)
    e2 = jnp.exp(s2 - s1)
    w1_ref[...] = 1.0 / (1.0 + e2)
    w2_ref[...] = e2 / (1.0 + e2)

    oh1 = (ids8 == a1)
    oh2 = (ids8 == a2)
    oh1f = oh1.astype(jnp.float32)
    oh2f = oh2.astype(jnp.float32)

    tri = (jax.lax.broadcasted_iota(jnp.int32, (T, T), 0)
           > jax.lax.broadcasted_iota(jnp.int32, (T, T), 1)
           ).astype(jnp.bfloat16)
    s1cnt = jnp.dot(tri, oh1.astype(jnp.bfloat16),
                    preferred_element_type=jnp.float32)
    s2cnt = jnp.dot(tri, oh2.astype(jnp.bfloat16),
                    preferred_element_type=jnp.float32)
    c0 = jnp.sum(oh1f, axis=0, keepdims=True)
    c1 = jnp.sum(oh2f, axis=0, keepdims=True)
    c = c0 + c1
    pc = jnp.ceil(c / _TB) * _TB
    triu8 = (jax.lax.broadcasted_iota(jnp.int32, (E, E), 0)
             < jax.lax.broadcasted_iota(jnp.int32, (E, E), 1)
             ).astype(jnp.float32)
    offs = jnp.dot(pc, triu8, preferred_element_type=jnp.float32)

    rank1 = jnp.sum(oh1f * s1cnt, axis=1, keepdims=True)
    rank2 = jnp.sum(oh2f * (s2cnt + c0), axis=1, keepdims=True)
    off1 = jnp.sum(oh1f * offs, axis=1, keepdims=True)
    off2 = jnp.sum(oh2f * offs, axis=1, keepdims=True)
    p1_ref[...] = (off1 + rank1).astype(jnp.int32)
    p2_ref[...] = (off2 + rank2).astype(jnp.int32)

    sb = (jax.lax.broadcasted_iota(jnp.int32, (64, 1), 0)
          .astype(jnp.float32) * _TB)
    gid = jnp.sum((offs <= sb).astype(jnp.float32), axis=1,
                  keepdims=True) - 1.0
    total = jnp.sum(pc, axis=1, keepdims=True)
    gidc = jnp.clip(gid, 0.0, E - 1.0)
    gid_ref[...] = gidc.astype(jnp.int32)
    validb = sb < total
    valid_ref[...] = validb.astype(jnp.int32)

    ids8b = jax.lax.broadcasted_iota(jnp.int32, (64, E), 1).astype(jnp.float32)
    ohg = (gidc == ids8b).astype(jnp.float32)
    offg = jnp.sum(ohg * offs, axis=1, keepdims=True)
    pcg = jnp.sum(ohg * pc, axis=1, keepdims=True)
    used = (c > 0.0).astype(jnp.float32)
    tri_incl = (jax.lax.broadcasted_iota(jnp.int32, (E, E), 0)
                <= jax.lax.broadcasted_iota(jnp.int32, (E, E), 1)
                ).astype(jnp.float32)
    cumu = jnp.dot(used, tri_incl,
                   preferred_element_type=jnp.float32)
    dist = jnp.sum(ohg * cumu, axis=1, keepdims=True) - 1.0
    par_ref[...] = (dist - 2.0 * jnp.floor(dist / 2.0)).astype(jnp.int32)
    firstb = (sb == offg) & validb
    first_ref[...] = firstb.astype(jnp.int32)
    end_sb = offg + pcg
    issue_ref[...] = (firstb & (end_sb < total)).astype(jnp.int32)
    ngid = jnp.sum((offs <= end_sb).astype(jnp.float32), axis=1,
                   keepdims=True) - 1.0
    ngid_ref[...] = jnp.clip(ngid, 0.0, E - 1.0).astype(jnp.int32)


def _expert_body(gid_sref, valid_sref, first_sref, issue_sref, ngid_sref,
                 par_sref, xb_ref, p1_ref, p2_ref,
                 wk_hbm, bk_ref, wv_hbm, bv_ref, y_ref,
                 wk_buf, wv_buf, sk0, sk1, sv0, sv1, *, T, D, H):
    b = pl.program_id(0)
    sks = (sk0, sk1)
    svs = (sv0, sv1)

    def _start(e_idx, slot):
        pltpu.make_async_copy(wk_hbm.at[e_idx], wk_buf.at[slot],
                              sks[slot]).start()
        pltpu.make_async_copy(wv_hbm.at[e_idx], wv_buf.at[slot],
                              svs[slot]).start()

    def _wait(slot):
        pltpu.make_async_copy(wk_hbm.at[0], wk_buf.at[slot],
                              sks[slot]).wait()
        pltpu.make_async_copy(wv_hbm.at[0], wv_buf.at[slot],
                              svs[slot]).wait()

    par_b = par_sref[b]

    @pl.when(b == 0)
    def _kickoff():
        _start(gid_sref[0], 0)

    for slot in (0, 1):
        @pl.when((issue_sref[b] == 1) & (par_b == 1 - slot))
        def _issue(slot=slot):
            _start(ngid_sref[b], slot)

    for slot in (0, 1):
        @pl.when((first_sref[b] == 1) & (par_b == slot))
        def _sync(slot=slot):
            _wait(slot)

    @pl.when(valid_sref[b] == 1)
    def _compute():
        slot = (jax.lax.broadcasted_iota(jnp.int32, (_TB, T), 0)
                + b * _TB)
        g = ((p1_ref[...] == slot) | (p2_ref[...] == slot)
             ).astype(jnp.bfloat16)
        xs = jnp.dot(g, xb_ref[...],
                     preferred_element_type=jnp.float32).astype(jnp.bfloat16)
        acc = jnp.zeros((_TB, D), dtype=jnp.float32)
        for c in range(H // _HB):
            sl = slice(c * _HB, (c + 1) * _HB)
            h = jax.nn.gelu(
                jnp.dot(xs, wk_buf[par_b, :, sl].astype(jnp.bfloat16),
                        preferred_element_type=jnp.float32)
                + bk_ref[0, :, sl])
            acc = acc + jnp.dot(h.astype(jnp.bfloat16),
                                wv_buf[par_b, sl, :].astype(jnp.bfloat16),
                                preferred_element_type=jnp.float32)
        y_ref[...] = (acc + bv_ref[0]).astype(jnp.bfloat16)

    @pl.when(valid_sref[b] == 0)
    def _pad():
        y_ref[...] = jnp.zeros_like(y_ref)


def _combine_body(xb_ref, p1_ref, p2_ref, w1_ref, w2_ref, y_ref,
                  wks_ref, bks_ref, wvs_ref, bvs_ref, out_ref, *, NS):
    hb = pl.program_id(1)

    @pl.when(hb == 0)
    def _combine():
        s_ids = jax.lax.broadcasted_iota(jnp.int32, (out_ref.shape[0], NS), 1)
        cb = (jnp.where(p1_ref[...] == s_ids, w1_ref[...], 0.0)
              + jnp.where(p2_ref[...] == s_ids, w2_ref[...], 0.0))
        out_ref[...] = (jnp.dot(cb.astype(jnp.bfloat16), y_ref[...],
                                preferred_element_type=jnp.float32)
                        + bvs_ref[...])

    hs = jax.nn.gelu(jnp.dot(xb_ref[...],
                             wks_ref[...].astype(jnp.bfloat16),
                             preferred_element_type=jnp.float32)
                     + bks_ref[...])
    out_ref[...] += jnp.dot(hs.astype(jnp.bfloat16),
                            wvs_ref[...].astype(jnp.bfloat16),
                            preferred_element_type=jnp.float32)


@functools.partial(jax.jit, static_argnames=())
def kernel(x, gate_kernel, Wk, bk, Wv, bv, Wk_s, bk_s, Wv_s, bv_s):
    B, S, D = x.shape
    T = B * S
    E = gate_kernel.shape[1]
    H = Wk.shape[2]
    NB = (2 * T) // _TB + E
    NS = NB * _TB
    NHB = H // _HB

    x2 = x.reshape(T, D)
    xb = x2.astype(jnp.bfloat16)
    bk3 = bk.reshape(E, 1, H)
    bv3 = bv.reshape(E, 1, D)
    bks2 = bk_s.reshape(1, H)
    bvs2 = bv_s.reshape(1, D)

    router = pl.pallas_call(
        functools.partial(_router_body, E=E, T=T, NB=NB),
        in_specs=[pl.BlockSpec((T, D), lambda: (0, 0)),
                  pl.BlockSpec((D, E), lambda: (0, 0))],
        out_specs=[pl.BlockSpec((T, 1), lambda: (0, 0)),
                   pl.BlockSpec((T, 1), lambda: (0, 0)),
                   pl.BlockSpec((T, 1), lambda: (0, 0)),
                   pl.BlockSpec((T, 1), lambda: (0, 0)),
                   pl.BlockSpec((64, 1), lambda: (0, 0)),
                   pl.BlockSpec((64, 1), lambda: (0, 0)),
                   pl.BlockSpec((64, 1), lambda: (0, 0)),
                   pl.BlockSpec((64, 1), lambda: (0, 0)),
                   pl.BlockSpec((64, 1), lambda: (0, 0)),
                   pl.BlockSpec((64, 1), lambda: (0, 0))],
        out_shape=[jax.ShapeDtypeStruct((T, 1), jnp.int32),
                   jax.ShapeDtypeStruct((T, 1), jnp.int32),
                   jax.ShapeDtypeStruct((T, 1), jnp.float32),
                   jax.ShapeDtypeStruct((T, 1), jnp.float32),
                   jax.ShapeDtypeStruct((64, 1), jnp.int32),
                   jax.ShapeDtypeStruct((64, 1), jnp.int32),
                   jax.ShapeDtypeStruct((64, 1), jnp.int32),
                   jax.ShapeDtypeStruct((64, 1), jnp.int32),
                   jax.ShapeDtypeStruct((64, 1), jnp.int32),
                   jax.ShapeDtypeStruct((64, 1), jnp.int32)],
    )(x2, gate_kernel)
    (p1c, p2c, w1c, w2c, gid64, valid64,
     first64, issue64, ngid64, par64) = router
    p1r = p1c.reshape(1, T)
    p2r = p2c.reshape(1, T)
    gids = gid64.reshape(64)[:NB]
    valid = valid64.reshape(64)[:NB]
    first = first64.reshape(64)[:NB]
    issue = issue64.reshape(64)[:NB]
    ngid = ngid64.reshape(64)[:NB]
    par = par64.reshape(64)[:NB]

    y = pl.pallas_call(
        functools.partial(_expert_body, T=T, D=D, H=H),
        grid_spec=pltpu.PrefetchScalarGridSpec(
            num_scalar_prefetch=6,
            grid=(NB,),
            in_specs=[
                pl.BlockSpec((T, D), lambda b, *s: (0, 0)),
                pl.BlockSpec((1, T), lambda b, *s: (0, 0)),
                pl.BlockSpec((1, T), lambda b, *s: (0, 0)),
                pl.BlockSpec(memory_space=pl.ANY),
                pl.BlockSpec((1, 1, H), lambda b, *s: (s[0][b], 0, 0)),
                pl.BlockSpec(memory_space=pl.ANY),
                pl.BlockSpec((1, 1, D), lambda b, *s: (s[0][b], 0, 0)),
            ],
            out_specs=pl.BlockSpec((_TB, D), lambda b, *s: (b, 0)),
            scratch_shapes=[
                pltpu.VMEM((2, D, H), jnp.float32),
                pltpu.VMEM((2, H, D), jnp.float32),
                pltpu.SemaphoreType.DMA,
                pltpu.SemaphoreType.DMA,
                pltpu.SemaphoreType.DMA,
                pltpu.SemaphoreType.DMA,
            ],
        ),
        out_shape=jax.ShapeDtypeStruct((NS, D), jnp.bfloat16),
        compiler_params=pltpu.CompilerParams(
            dimension_semantics=("arbitrary",)),
    )(gids, valid, first, issue, ngid, par, xb, p1r, p2r, Wk, bk3, Wv, bv3)

    TBC = 256
    out = pl.pallas_call(
        functools.partial(_combine_body, NS=NS),
        grid=(T // TBC, NHB),
        in_specs=[
            pl.BlockSpec((TBC, D), lambda t, h: (t, 0)),
            pl.BlockSpec((TBC, 1), lambda t, h: (t, 0)),
            pl.BlockSpec((TBC, 1), lambda t, h: (t, 0)),
            pl.BlockSpec((TBC, 1), lambda t, h: (t, 0)),
            pl.BlockSpec((TBC, 1), lambda t, h: (t, 0)),
            pl.BlockSpec((NS, D), lambda t, h: (0, 0)),
            pl.BlockSpec((D, _HB), lambda t, h: (0, h)),
            pl.BlockSpec((1, _HB), lambda t, h: (0, h)),
            pl.BlockSpec((_HB, D), lambda t, h: (h, 0)),
            pl.BlockSpec((1, D), lambda t, h: (0, 0)),
        ],
        out_specs=pl.BlockSpec((TBC, D), lambda t, h: (t, 0)),
        out_shape=jax.ShapeDtypeStruct((T, D), jnp.float32),
        compiler_params=pltpu.CompilerParams(
            dimension_semantics=("arbitrary", "arbitrary")),
    )(xb, p1c, p2c, w1c, w2c, y, Wk_s, bks2, Wv_s, bvs2)

    return (out.reshape(B, S, D), jnp.float32(0.0))


# speedup vs baseline: 1.1870x; 1.0269x over previous
import functools

import jax
import jax.numpy as jnp
from jax.experimental import pallas as pl
from jax.experimental.pallas import tpu as pltpu

_TB = 256
_HB = 512


def _router_body(x_ref, gate_ref,
                 p1_ref, p2_ref, w1_ref, w2_ref, gid_ref, valid_ref,
                 first_ref, issue_ref, ngid_ref, par_ref,
                 *, E, T, NB):
    logits = jnp.dot(x_ref[...], gate_ref[...],
                     preferred_element_type=jnp.float32)
    ids8 = jax.lax.broadcasted_iota(jnp.int32, (T, E), 1)
    a1 = jnp.argmax(logits, axis=1, keepdims=True)
    s1 = jnp.max(logits, axis=1, keepdims=True)
    masked = jnp.where(ids8 == a1, -jnp.inf, logits)
    a2 = jnp.argmax(masked, axis=1, keepdims=True)
    s2 = jnp.max(masked, axis=1, keepdims=True)
    e2 = jnp.exp(s2 - s1)
    w1_ref[...] = 1.0 / (1.0 + e2)
    w2_ref[...] = e2 / (1.0 + e2)

    oh1 = (ids8 == a1)
    oh2 = (ids8 == a2)
    oh1f = oh1.astype(jnp.float32)
    oh2f = oh2.astype(jnp.float32)

    tri = (jax.lax.broadcasted_iota(jnp.int32, (T, T), 0)
           > jax.lax.broadcasted_iota(jnp.int32, (T, T), 1)
           ).astype(jnp.bfloat16)
    s1cnt = jnp.dot(tri, oh1.astype(jnp.bfloat16),
                    preferred_element_type=jnp.float32)
    s2cnt = jnp.dot(tri, oh2.astype(jnp.bfloat16),
                    preferred_element_type=jnp.float32)
    c0 = jnp.sum(oh1f, axis=0, keepdims=True)
    c1 = jnp.sum(oh2f, axis=0, keepdims=True)
    c = c0 + c1
    pc = jnp.ceil(c / _TB) * _TB
    triu8 = (jax.lax.broadcasted_iota(jnp.int32, (E, E), 0)
             < jax.lax.broadcasted_iota(jnp.int32, (E, E), 1)
             ).astype(jnp.float32)
    offs = jnp.dot(pc, triu8, preferred_element_type=jnp.float32)

    rank1 = jnp.sum(oh1f * s1cnt, axis=1, keepdims=True)
    rank2 = jnp.sum(oh2f * (s2cnt + c0), axis=1, keepdims=True)
    off1 = jnp.sum(oh1f * offs, axis=1, keepdims=True)
    off2 = jnp.sum(oh2f * offs, axis=1, keepdims=True)
    p1_ref[...] = (off1 + rank1).astype(jnp.int32)
    p2_ref[...] = (off2 + rank2).astype(jnp.int32)

    sb = (jax.lax.broadcasted_iota(jnp.int32, (64, 1), 0)
          .astype(jnp.float32) * _TB)
    gid = jnp.sum((offs <= sb).astype(jnp.float32), axis=1,
                  keepdims=True) - 1.0
    total = jnp.sum(pc, axis=1, keepdims=True)
    gidc = jnp.clip(gid, 0.0, E - 1.0)
    gid_ref[...] = gidc.astype(jnp.int32)
    validb = sb < total
    valid_ref[...] = validb.astype(jnp.int32)

    ids8b = jax.lax.broadcasted_iota(jnp.int32, (64, E), 1).astype(jnp.float32)
    ohg = (gidc == ids8b).astype(jnp.float32)
    offg = jnp.sum(ohg * offs, axis=1, keepdims=True)
    pcg = jnp.sum(ohg * pc, axis=1, keepdims=True)
    used = (c > 0.0).astype(jnp.float32)
    tri_incl = (jax.lax.broadcasted_iota(jnp.int32, (E, E), 0)
                <= jax.lax.broadcasted_iota(jnp.int32, (E, E), 1)
                ).astype(jnp.float32)
    cumu = jnp.dot(used, tri_incl,
                   preferred_element_type=jnp.float32)
    dist = jnp.sum(ohg * cumu, axis=1, keepdims=True) - 1.0
    par_ref[...] = (dist - 2.0 * jnp.floor(dist / 2.0)).astype(jnp.int32)
    firstb = (sb == offg) & validb
    first_ref[...] = firstb.astype(jnp.int32)
    end_sb = offg + pcg
    issue_ref[...] = (firstb & (end_sb < total)).astype(jnp.int32)
    ngid = jnp.sum((offs <= end_sb).astype(jnp.float32), axis=1,
                   keepdims=True) - 1.0
    ngid_ref[...] = jnp.clip(ngid, 0.0, E - 1.0).astype(jnp.int32)


def _expert_body(gid_sref, valid_sref, first_sref, issue_sref, ngid_sref,
                 par_sref, xb_ref, p1_ref, p2_ref,
                 p1c_ref, p2c_ref, w1c_ref, w2c_ref,
                 wk_hbm, bk_ref, wv_hbm, bv_ref, moe_ref,
                 wk_buf, wv_buf, sk0, sk1, sv0, sv1, *, T, D, H):
    b = pl.program_id(0)
    sks = (sk0, sk1)
    svs = (sv0, sv1)

    def _start(e_idx, slot):
        pltpu.make_async_copy(wk_hbm.at[e_idx], wk_buf.at[slot],
                              sks[slot]).start()
        pltpu.make_async_copy(wv_hbm.at[e_idx], wv_buf.at[slot],
                              svs[slot]).start()

    def _wait(slot):
        pltpu.make_async_copy(wk_hbm.at[0], wk_buf.at[slot],
                              sks[slot]).wait()
        pltpu.make_async_copy(wv_hbm.at[0], wv_buf.at[slot],
                              svs[slot]).wait()

    par_b = par_sref[b]

    @pl.when(b == 0)
    def _kickoff():
        _start(gid_sref[0], 0)
        moe_ref[...] = jnp.zeros_like(moe_ref)

    for slot in (0, 1):
        @pl.when((issue_sref[b] == 1) & (par_b == 1 - slot))
        def _issue(slot=slot):
            _start(ngid_sref[b], slot)

    for slot in (0, 1):
        @pl.when((first_sref[b] == 1) & (par_b == slot))
        def _sync(slot=slot):
            _wait(slot)

    @pl.when(valid_sref[b] == 1)
    def _compute():
        slot = (jax.lax.broadcasted_iota(jnp.int32, (_TB, T), 0)
                + b * _TB)
        g = ((p1_ref[...] == slot) | (p2_ref[...] == slot)
             ).astype(jnp.bfloat16)
        xs = jnp.dot(g, xb_ref[...],
                     preferred_element_type=jnp.float32).astype(jnp.bfloat16)
        acc = jnp.zeros((_TB, D), dtype=jnp.float32)
        for c in range(H // _HB):
            sl = slice(c * _HB, (c + 1) * _HB)
            h = jax.nn.gelu(
                jnp.dot(xs, wk_buf[par_b, :, sl].astype(jnp.bfloat16),
                        preferred_element_type=jnp.float32)
                + bk_ref[0, :, sl])
            acc = acc + jnp.dot(h.astype(jnp.bfloat16),
                                wv_buf[par_b, sl, :].astype(jnp.bfloat16),
                                preferred_element_type=jnp.float32)
        yb = (acc + bv_ref[0]).astype(jnp.bfloat16)
        slot_l = (jax.lax.broadcasted_iota(jnp.int32, (T, _TB), 1)
                  + b * _TB)
        cw = (jnp.where(p1c_ref[...] == slot_l, w1c_ref[...], 0.0)
              + jnp.where(p2c_ref[...] == slot_l, w2c_ref[...], 0.0))
        moe_ref[...] += jnp.dot(cw.astype(jnp.bfloat16), yb,
                                preferred_element_type=jnp.float32)


def _combine_body(xb_ref, moe_ref,
                  wks_ref, bks_ref, wvs_ref, bvs_ref, out_ref):
    hb = pl.program_id(1)

    @pl.when(hb == 0)
    def _init():
        out_ref[...] = moe_ref[...] + bvs_ref[...]

    hs = jax.nn.gelu(jnp.dot(xb_ref[...],
                             wks_ref[...].astype(jnp.bfloat16),
                             preferred_element_type=jnp.float32)
                     + bks_ref[...])
    out_ref[...] += jnp.dot(hs.astype(jnp.bfloat16),
                            wvs_ref[...].astype(jnp.bfloat16),
                            preferred_element_type=jnp.float32)


@functools.partial(jax.jit, static_argnames=())
def kernel(x, gate_kernel, Wk, bk, Wv, bv, Wk_s, bk_s, Wv_s, bv_s):
    B, S, D = x.shape
    T = B * S
    E = gate_kernel.shape[1]
    H = Wk.shape[2]
    NB = (2 * T) // _TB + E
    NS = NB * _TB
    NHB = H // _HB

    x2 = x.reshape(T, D)
    xb = x2.astype(jnp.bfloat16)
    bk3 = bk.reshape(E, 1, H)
    bv3 = bv.reshape(E, 1, D)
    bks2 = bk_s.reshape(1, H)
    bvs2 = bv_s.reshape(1, D)

    router = pl.pallas_call(
        functools.partial(_router_body, E=E, T=T, NB=NB),
        in_specs=[pl.BlockSpec((T, D), lambda: (0, 0)),
                  pl.BlockSpec((D, E), lambda: (0, 0))],
        out_specs=[pl.BlockSpec((T, 1), lambda: (0, 0)),
                   pl.BlockSpec((T, 1), lambda: (0, 0)),
                   pl.BlockSpec((T, 1), lambda: (0, 0)),
                   pl.BlockSpec((T, 1), lambda: (0, 0)),
                   pl.BlockSpec((64, 1), lambda: (0, 0)),
                   pl.BlockSpec((64, 1), lambda: (0, 0)),
                   pl.BlockSpec((64, 1), lambda: (0, 0)),
                   pl.BlockSpec((64, 1), lambda: (0, 0)),
                   pl.BlockSpec((64, 1), lambda: (0, 0)),
                   pl.BlockSpec((64, 1), lambda: (0, 0))],
        out_shape=[jax.ShapeDtypeStruct((T, 1), jnp.int32),
                   jax.ShapeDtypeStruct((T, 1), jnp.int32),
                   jax.ShapeDtypeStruct((T, 1), jnp.float32),
                   jax.ShapeDtypeStruct((T, 1), jnp.float32),
                   jax.ShapeDtypeStruct((64, 1), jnp.int32),
                   jax.ShapeDtypeStruct((64, 1), jnp.int32),
                   jax.ShapeDtypeStruct((64, 1), jnp.int32),
                   jax.ShapeDtypeStruct((64, 1), jnp.int32),
                   jax.ShapeDtypeStruct((64, 1), jnp.int32),
                   jax.ShapeDtypeStruct((64, 1), jnp.int32)],
    )(x2, gate_kernel)
    (p1c, p2c, w1c, w2c, gid64, valid64,
     first64, issue64, ngid64, par64) = router
    p1r = p1c.reshape(1, T)
    p2r = p2c.reshape(1, T)
    gids = gid64.reshape(64)[:NB]
    valid = valid64.reshape(64)[:NB]
    first = first64.reshape(64)[:NB]
    issue = issue64.reshape(64)[:NB]
    ngid = ngid64.reshape(64)[:NB]
    par = par64.reshape(64)[:NB]

    moe = pl.pallas_call(
        functools.partial(_expert_body, T=T, D=D, H=H),
        grid_spec=pltpu.PrefetchScalarGridSpec(
            num_scalar_prefetch=6,
            grid=(NB,),
            in_specs=[
                pl.BlockSpec((T, D), lambda b, *s: (0, 0)),
                pl.BlockSpec((1, T), lambda b, *s: (0, 0)),
                pl.BlockSpec((1, T), lambda b, *s: (0, 0)),
                pl.BlockSpec((T, 1), lambda b, *s: (0, 0)),
                pl.BlockSpec((T, 1), lambda b, *s: (0, 0)),
                pl.BlockSpec((T, 1), lambda b, *s: (0, 0)),
                pl.BlockSpec((T, 1), lambda b, *s: (0, 0)),
                pl.BlockSpec(memory_space=pl.ANY),
                pl.BlockSpec((1, 1, H), lambda b, *s: (s[0][b], 0, 0)),
                pl.BlockSpec(memory_space=pl.ANY),
                pl.BlockSpec((1, 1, D), lambda b, *s: (s[0][b], 0, 0)),
            ],
            out_specs=pl.BlockSpec((T, D), lambda b, *s: (0, 0)),
            scratch_shapes=[
                pltpu.VMEM((2, D, H), jnp.float32),
                pltpu.VMEM((2, H, D), jnp.float32),
                pltpu.SemaphoreType.DMA,
                pltpu.SemaphoreType.DMA,
                pltpu.SemaphoreType.DMA,
                pltpu.SemaphoreType.DMA,
            ],
        ),
        out_shape=jax.ShapeDtypeStruct((T, D), jnp.float32),
        compiler_params=pltpu.CompilerParams(
            dimension_semantics=("arbitrary",)),
    )(gids, valid, first, issue, ngid, par, xb, p1r, p2r,
      p1c, p2c, w1c, w2c, Wk, bk3, Wv, bv3)

    TBC = 256
    out = pl.pallas_call(
        _combine_body,
        grid=(T // TBC, NHB),
        in_specs=[
            pl.BlockSpec((TBC, D), lambda t, h: (t, 0)),
            pl.BlockSpec((TBC, D), lambda t, h: (t, 0)),
            pl.BlockSpec((D, _HB), lambda t, h: (0, h)),
            pl.BlockSpec((1, _HB), lambda t, h: (0, h)),
            pl.BlockSpec((_HB, D), lambda t, h: (h, 0)),
            pl.BlockSpec((1, D), lambda t, h: (0, 0)),
        ],
        out_specs=pl.BlockSpec((TBC, D), lambda t, h: (t, 0)),
        out_shape=jax.ShapeDtypeStruct((T, D), jnp.float32),
        compiler_params=pltpu.CompilerParams(
            dimension_semantics=("arbitrary", "arbitrary")),
    )(xb, moe, Wk_s, bks2, Wv_s, bvs2)

    return (out.reshape(B, S, D), jnp.float32(0.0))


# speedup vs baseline: 1.4375x; 1.2110x over previous
import functools

import jax
import jax.numpy as jnp
from jax.experimental import pallas as pl
from jax.experimental.pallas import tpu as pltpu

_TB = 256
_HB = 512


def _router_body(x_ref, gate_ref,
                 p1_ref, p2_ref, w1_ref, w2_ref, gid_ref, valid_ref,
                 first_ref, issue_ref, ngid_ref, par_ref,
                 *, E, T, NB):
    logits = jnp.dot(x_ref[...], gate_ref[...],
                     preferred_element_type=jnp.float32)
    ids8 = jax.lax.broadcasted_iota(jnp.int32, (T, E), 1)
    a1 = jnp.argmax(logits, axis=1, keepdims=True)
    s1 = jnp.max(logits, axis=1, keepdims=True)
    masked = jnp.where(ids8 == a1, -jnp.inf, logits)
    a2 = jnp.argmax(masked, axis=1, keepdims=True)
    s2 = jnp.max(masked, axis=1, keepdims=True)
    e2 = jnp.exp(s2 - s1)
    w1_ref[...] = 1.0 / (1.0 + e2)
    w2_ref[...] = e2 / (1.0 + e2)

    oh1 = (ids8 == a1)
    oh2 = (ids8 == a2)
    oh1f = oh1.astype(jnp.float32)
    oh2f = oh2.astype(jnp.float32)

    tri = (jax.lax.broadcasted_iota(jnp.int32, (T, T), 0)
           > jax.lax.broadcasted_iota(jnp.int32, (T, T), 1)
           ).astype(jnp.bfloat16)
    s1cnt = jnp.dot(tri, oh1.astype(jnp.bfloat16),
                    preferred_element_type=jnp.float32)
    s2cnt = jnp.dot(tri, oh2.astype(jnp.bfloat16),
                    preferred_element_type=jnp.float32)
    c0 = jnp.sum(oh1f, axis=0, keepdims=True)
    c1 = jnp.sum(oh2f, axis=0, keepdims=True)
    c = c0 + c1
    pc = jnp.ceil(c / _TB) * _TB
    triu8 = (jax.lax.broadcasted_iota(jnp.int32, (E, E), 0)
             < jax.lax.broadcasted_iota(jnp.int32, (E, E), 1)
             ).astype(jnp.float32)
    offs = jnp.dot(pc, triu8, preferred_element_type=jnp.float32)

    rank1 = jnp.sum(oh1f * s1cnt, axis=1, keepdims=True)
    rank2 = jnp.sum(oh2f * (s2cnt + c0), axis=1, keepdims=True)
    off1 = jnp.sum(oh1f * offs, axis=1, keepdims=True)
    off2 = jnp.sum(oh2f * offs, axis=1, keepdims=True)
    p1_ref[...] = (off1 + rank1).astype(jnp.int32)
    p2_ref[...] = (off2 + rank2).astype(jnp.int32)

    sb = (jax.lax.broadcasted_iota(jnp.int32, (64, 1), 0)
          .astype(jnp.float32) * _TB)
    gid = jnp.sum((offs <= sb).astype(jnp.float32), axis=1,
                  keepdims=True) - 1.0
    total = jnp.sum(pc, axis=1, keepdims=True)
    gidc = jnp.clip(gid, 0.0, E - 1.0)
    gid_ref[...] = gidc.astype(jnp.int32)
    validb = sb < total
    valid_ref[...] = validb.astype(jnp.int32)

    ids8b = jax.lax.broadcasted_iota(jnp.int32, (64, E), 1).astype(jnp.float32)
    ohg = (gidc == ids8b).astype(jnp.float32)
    offg = jnp.sum(ohg * offs, axis=1, keepdims=True)
    pcg = jnp.sum(ohg * pc, axis=1, keepdims=True)
    used = (c > 0.0).astype(jnp.float32)
    tri_incl = (jax.lax.broadcasted_iota(jnp.int32, (E, E), 0)
                <= jax.lax.broadcasted_iota(jnp.int32, (E, E), 1)
                ).astype(jnp.float32)
    cumu = jnp.dot(used, tri_incl,
                   preferred_element_type=jnp.float32)
    dist = jnp.sum(ohg * cumu, axis=1, keepdims=True) - 1.0
    par_ref[...] = (dist - 2.0 * jnp.floor(dist / 2.0)).astype(jnp.int32)
    firstb = (sb == offg) & validb
    first_ref[...] = firstb.astype(jnp.int32)
    end_sb = offg + pcg
    issue_ref[...] = (firstb & (end_sb < total)).astype(jnp.int32)
    ngid = jnp.sum((offs <= end_sb).astype(jnp.float32), axis=1,
                   keepdims=True) - 1.0
    ngid_ref[...] = jnp.clip(ngid, 0.0, E - 1.0).astype(jnp.int32)


def _expert_body(gid_sref, valid_sref, first_sref, issue_sref, ngid_sref,
                 par_sref, xb_ref, p1_ref, p2_ref,
                 p1c_ref, p2c_ref, w1c_ref, w2c_ref,
                 wk_hbm, bk_ref, wv_hbm, bv_ref, moe_ref,
                 wk_buf, wv_buf, sk0, sk1, sv0, sv1, *, T, D, H):
    b = pl.program_id(0)
    sks = (sk0, sk1)
    svs = (sv0, sv1)

    def _start(e_idx, slot):
        pltpu.make_async_copy(wk_hbm.at[e_idx], wk_buf.at[slot],
                              sks[slot]).start()
        pltpu.make_async_copy(wv_hbm.at[e_idx], wv_buf.at[slot],
                              svs[slot]).start()

    def _wait(slot):
        pltpu.make_async_copy(wk_hbm.at[0], wk_buf.at[slot],
                              sks[slot]).wait()
        pltpu.make_async_copy(wv_hbm.at[0], wv_buf.at[slot],
                              svs[slot]).wait()

    par_b = par_sref[b]

    @pl.when(b == 0)
    def _kickoff():
        _start(gid_sref[0], 0)
        moe_ref[...] = jnp.zeros_like(moe_ref)

    for slot in (0, 1):
        @pl.when((issue_sref[b] == 1) & (par_b == 1 - slot))
        def _issue(slot=slot):
            _start(ngid_sref[b], slot)

    for slot in (0, 1):
        @pl.when((first_sref[b] == 1) & (par_b == slot))
        def _sync(slot=slot):
            _wait(slot)

    @pl.when(valid_sref[b] == 1)
    def _compute():
        slot = (jax.lax.broadcasted_iota(jnp.int32, (_TB, T), 0)
                + b * _TB)
        g = ((p1_ref[...] == slot) | (p2_ref[...] == slot)
             ).astype(jnp.bfloat16)
        xs = jnp.dot(g, xb_ref[...],
                     preferred_element_type=jnp.float32).astype(jnp.bfloat16)
        acc = jnp.zeros((_TB, D), dtype=jnp.float32)
        for c in range(H // _HB):
            sl = slice(c * _HB, (c + 1) * _HB)
            h = jax.nn.gelu(
                jnp.dot(xs, wk_buf[par_b, :, sl].astype(jnp.bfloat16),
                        preferred_element_type=jnp.float32)
                + bk_ref[0, :, sl])
            acc = acc + jnp.dot(h.astype(jnp.bfloat16),
                                wv_buf[par_b, sl, :].astype(jnp.bfloat16),
                                preferred_element_type=jnp.float32)
        yb = (acc + bv_ref[0]).astype(jnp.bfloat16)
        slot_l = (jax.lax.broadcasted_iota(jnp.int32, (T, _TB), 1)
                  + b * _TB)
        cw = (jnp.where(p1c_ref[...] == slot_l, w1c_ref[...], 0.0)
              + jnp.where(p2c_ref[...] == slot_l, w2c_ref[...], 0.0))
        moe_ref[...] += jnp.dot(cw.astype(jnp.bfloat16), yb,
                                preferred_element_type=jnp.float32)


def _combine_body(xb_ref, moe_ref,
                  wks_ref, bks_ref, wvs_ref, bvs_ref, out_ref):
    hb = pl.program_id(0)

    @pl.when(hb == 0)
    def _init():
        out_ref[...] = moe_ref[...] + bvs_ref[...]

    hs = jax.nn.gelu(jnp.dot(xb_ref[...],
                             wks_ref[...].astype(jnp.bfloat16),
                             preferred_element_type=jnp.float32)
                     + bks_ref[...])
    out_ref[...] += jnp.dot(hs.astype(jnp.bfloat16),
                            wvs_ref[...].astype(jnp.bfloat16),
                            preferred_element_type=jnp.float32)


@functools.partial(jax.jit, static_argnames=())
def kernel(x, gate_kernel, Wk, bk, Wv, bv, Wk_s, bk_s, Wv_s, bv_s):
    B, S, D = x.shape
    T = B * S
    E = gate_kernel.shape[1]
    H = Wk.shape[2]
    NB = (2 * T) // _TB + E
    NS = NB * _TB
    NHB = H // _HB

    x2 = x.reshape(T, D)
    xb = x2.astype(jnp.bfloat16)
    bk3 = bk.reshape(E, 1, H)
    bv3 = bv.reshape(E, 1, D)
    bks2 = bk_s.reshape(1, H)
    bvs2 = bv_s.reshape(1, D)

    router = pl.pallas_call(
        functools.partial(_router_body, E=E, T=T, NB=NB),
        in_specs=[pl.BlockSpec((T, D), lambda: (0, 0)),
                  pl.BlockSpec((D, E), lambda: (0, 0))],
        out_specs=[pl.BlockSpec((T, 1), lambda: (0, 0)),
                   pl.BlockSpec((T, 1), lambda: (0, 0)),
                   pl.BlockSpec((T, 1), lambda: (0, 0)),
                   pl.BlockSpec((T, 1), lambda: (0, 0)),
                   pl.BlockSpec((64, 1), lambda: (0, 0)),
                   pl.BlockSpec((64, 1), lambda: (0, 0)),
                   pl.BlockSpec((64, 1), lambda: (0, 0)),
                   pl.BlockSpec((64, 1), lambda: (0, 0)),
                   pl.BlockSpec((64, 1), lambda: (0, 0)),
                   pl.BlockSpec((64, 1), lambda: (0, 0))],
        out_shape=[jax.ShapeDtypeStruct((T, 1), jnp.int32),
                   jax.ShapeDtypeStruct((T, 1), jnp.int32),
                   jax.ShapeDtypeStruct((T, 1), jnp.float32),
                   jax.ShapeDtypeStruct((T, 1), jnp.float32),
                   jax.ShapeDtypeStruct((64, 1), jnp.int32),
                   jax.ShapeDtypeStruct((64, 1), jnp.int32),
                   jax.ShapeDtypeStruct((64, 1), jnp.int32),
                   jax.ShapeDtypeStruct((64, 1), jnp.int32),
                   jax.ShapeDtypeStruct((64, 1), jnp.int32),
                   jax.ShapeDtypeStruct((64, 1), jnp.int32)],
    )(x2, gate_kernel)
    (p1c, p2c, w1c, w2c, gid64, valid64,
     first64, issue64, ngid64, par64) = router
    p1r = p1c.reshape(1, T)
    p2r = p2c.reshape(1, T)
    gids = gid64.reshape(64)[:NB]
    valid = valid64.reshape(64)[:NB]
    first = first64.reshape(64)[:NB]
    issue = issue64.reshape(64)[:NB]
    ngid = ngid64.reshape(64)[:NB]
    par = par64.reshape(64)[:NB]

    moe = pl.pallas_call(
        functools.partial(_expert_body, T=T, D=D, H=H),
        grid_spec=pltpu.PrefetchScalarGridSpec(
            num_scalar_prefetch=6,
            grid=(NB,),
            in_specs=[
                pl.BlockSpec((T, D), lambda b, *s: (0, 0)),
                pl.BlockSpec((1, T), lambda b, *s: (0, 0)),
                pl.BlockSpec((1, T), lambda b, *s: (0, 0)),
                pl.BlockSpec((T, 1), lambda b, *s: (0, 0)),
                pl.BlockSpec((T, 1), lambda b, *s: (0, 0)),
                pl.BlockSpec((T, 1), lambda b, *s: (0, 0)),
                pl.BlockSpec((T, 1), lambda b, *s: (0, 0)),
                pl.BlockSpec(memory_space=pl.ANY),
                pl.BlockSpec((1, 1, H), lambda b, *s: (s[0][b], 0, 0)),
                pl.BlockSpec(memory_space=pl.ANY),
                pl.BlockSpec((1, 1, D), lambda b, *s: (s[0][b], 0, 0)),
            ],
            out_specs=pl.BlockSpec((T, D), lambda b, *s: (0, 0)),
            scratch_shapes=[
                pltpu.VMEM((2, D, H), jnp.float32),
                pltpu.VMEM((2, H, D), jnp.float32),
                pltpu.SemaphoreType.DMA,
                pltpu.SemaphoreType.DMA,
                pltpu.SemaphoreType.DMA,
                pltpu.SemaphoreType.DMA,
            ],
        ),
        out_shape=jax.ShapeDtypeStruct((T, D), jnp.float32),
        compiler_params=pltpu.CompilerParams(
            dimension_semantics=("arbitrary",)),
    )(gids, valid, first, issue, ngid, par, xb, p1r, p2r,
      p1c, p2c, w1c, w2c, Wk, bk3, Wv, bv3)

    out = pl.pallas_call(
        _combine_body,
        grid=(NHB,),
        in_specs=[
            pl.BlockSpec((T, D), lambda h: (0, 0)),
            pl.BlockSpec((T, D), lambda h: (0, 0)),
            pl.BlockSpec((D, _HB), lambda h: (0, h)),
            pl.BlockSpec((1, _HB), lambda h: (0, h)),
            pl.BlockSpec((_HB, D), lambda h: (h, 0)),
            pl.BlockSpec((1, D), lambda h: (0, 0)),
        ],
        out_specs=pl.BlockSpec((T, D), lambda h: (0, 0)),
        out_shape=jax.ShapeDtypeStruct((T, D), jnp.float32),
        compiler_params=pltpu.CompilerParams(
            dimension_semantics=("arbitrary",)),
    )(xb, moe, Wk_s, bks2, Wv_s, bvs2)

    return (out.reshape(B, S, D), jnp.float32(0.0))


# speedup vs baseline: 1.4395x; 1.0014x over previous
import functools

import jax
import jax.numpy as jnp
from jax.experimental import pallas as pl
from jax.experimental.pallas import tpu as pltpu

_TB = 256
_HB = 512


def _router_body(x_ref, gate_ref,
                 p1_ref, p2_ref, w1_ref, w2_ref, gid_ref, valid_ref,
                 first_ref, issue_ref, ngid_ref, par_ref, spar_ref,
                 *, E, T, NB):
    logits = jnp.dot(x_ref[...], gate_ref[...],
                     preferred_element_type=jnp.float32)
    ids8 = jax.lax.broadcasted_iota(jnp.int32, (T, E), 1)
    a1 = jnp.argmax(logits, axis=1, keepdims=True)
    s1 = jnp.max(logits, axis=1, keepdims=True)
    masked = jnp.where(ids8 == a1, -jnp.inf, logits)
    a2 = jnp.argmax(masked, axis=1, keepdims=True)
    s2 = jnp.max(masked, axis=1, keepdims=True)
    e2 = jnp.exp(s2 - s1)
    w1_ref[...] = 1.0 / (1.0 + e2)
    w2_ref[...] = e2 / (1.0 + e2)

    oh1 = (ids8 == a1)
    oh2 = (ids8 == a2)
    oh1f = oh1.astype(jnp.float32)
    oh2f = oh2.astype(jnp.float32)

    tri = (jax.lax.broadcasted_iota(jnp.int32, (T, T), 0)
           > jax.lax.broadcasted_iota(jnp.int32, (T, T), 1)
           ).astype(jnp.bfloat16)
    s1cnt = jnp.dot(tri, oh1.astype(jnp.bfloat16),
                    preferred_element_type=jnp.float32)
    s2cnt = jnp.dot(tri, oh2.astype(jnp.bfloat16),
                    preferred_element_type=jnp.float32)
    c0 = jnp.sum(oh1f, axis=0, keepdims=True)
    c1 = jnp.sum(oh2f, axis=0, keepdims=True)
    c = c0 + c1
    pc = jnp.ceil(c / _TB) * _TB
    triu8 = (jax.lax.broadcasted_iota(jnp.int32, (E, E), 0)
             < jax.lax.broadcasted_iota(jnp.int32, (E, E), 1)
             ).astype(jnp.float32)
    offs = jnp.dot(pc, triu8, preferred_element_type=jnp.float32)

    rank1 = jnp.sum(oh1f * s1cnt, axis=1, keepdims=True)
    rank2 = jnp.sum(oh2f * (s2cnt + c0), axis=1, keepdims=True)
    off1 = jnp.sum(oh1f * offs, axis=1, keepdims=True)
    off2 = jnp.sum(oh2f * offs, axis=1, keepdims=True)
    p1_ref[...] = (off1 + rank1).astype(jnp.int32)
    p2_ref[...] = (off2 + rank2).astype(jnp.int32)

    sb = (jax.lax.broadcasted_iota(jnp.int32, (64, 1), 0)
          .astype(jnp.float32) * _TB)
    gid = jnp.sum((offs <= sb).astype(jnp.float32), axis=1,
                  keepdims=True) - 1.0
    total = jnp.sum(pc, axis=1, keepdims=True)
    gidc = jnp.clip(gid, 0.0, E - 1.0)
    gid_ref[...] = gidc.astype(jnp.int32)
    validb = sb < total
    valid_ref[...] = validb.astype(jnp.int32)

    ids8b = jax.lax.broadcasted_iota(jnp.int32, (64, E), 1).astype(jnp.float32)
    ohg = (gidc == ids8b).astype(jnp.float32)
    offg = jnp.sum(ohg * offs, axis=1, keepdims=True)
    pcg = jnp.sum(ohg * pc, axis=1, keepdims=True)
    used = (c > 0.0).astype(jnp.float32)
    tri_incl = (jax.lax.broadcasted_iota(jnp.int32, (E, E), 0)
                <= jax.lax.broadcasted_iota(jnp.int32, (E, E), 1)
                ).astype(jnp.float32)
    cumu = jnp.dot(used, tri_incl,
                   preferred_element_type=jnp.float32)
    dist = jnp.sum(ohg * cumu, axis=1, keepdims=True) - 1.0
    par_ref[...] = (dist - 2.0 * jnp.floor(dist / 2.0)).astype(jnp.int32)
    firstb = (sb == offg) & validb
    first_ref[...] = firstb.astype(jnp.int32)
    end_sb = offg + pcg
    issue_ref[...] = firstb.astype(jnp.int32)
    ngid = jnp.sum((offs <= end_sb).astype(jnp.float32), axis=1,
                   keepdims=True) - 1.0
    ngid_ref[...] = jnp.where(end_sb < total,
                              jnp.clip(ngid, 0.0, E - 1.0),
                              float(E)).astype(jnp.int32)
    nuse = jnp.sum(used, axis=1, keepdims=True)
    spar = nuse - 2.0 * jnp.floor(nuse / 2.0)
    spar_ref[...] = jnp.broadcast_to(spar, (64, 1)).astype(jnp.int32)


def _expert_body(gid_sref, valid_sref, first_sref, issue_sref, ngid_sref,
                 par_sref, spar_sref, xb_ref, p1_ref, p2_ref,
                 p1c_ref, p2c_ref, w1c_ref, w2c_ref,
                 wk_hbm, bk_ref, wv_hbm, bv_ref,
                 wks_hbm, wvs_hbm, bks_ref, bvs_ref, moe_ref,
                 wk_buf, wv_buf, sk0, sk1, sv0, sv1, *, T, D, H, E, NB):
    b = pl.program_id(0)
    sks = (sk0, sk1)
    svs = (sv0, sv1)

    def _start(e_idx, slot):
        @pl.when(e_idx < E)
        def _expert_w():
            pltpu.make_async_copy(wk_hbm.at[e_idx], wk_buf.at[slot],
                                  sks[slot]).start()
            pltpu.make_async_copy(wv_hbm.at[e_idx], wv_buf.at[slot],
                                  svs[slot]).start()

        @pl.when(e_idx == E)
        def _shared_w():
            pltpu.make_async_copy(wks_hbm, wk_buf.at[slot],
                                  sks[slot]).start()
            pltpu.make_async_copy(wvs_hbm, wv_buf.at[slot],
                                  svs[slot]).start()

    def _wait(slot):
        pltpu.make_async_copy(wk_hbm.at[0], wk_buf.at[slot],
                              sks[slot]).wait()
        pltpu.make_async_copy(wv_hbm.at[0], wv_buf.at[slot],
                              svs[slot]).wait()

    par_b = par_sref[b]

    @pl.when(b == 0)
    def _kickoff():
        _start(gid_sref[0], 0)
        moe_ref[...] = jnp.zeros_like(moe_ref)

    for slot in (0, 1):
        @pl.when((b < NB) & (issue_sref[b] == 1) & (par_b == 1 - slot))
        def _issue(slot=slot):
            _start(ngid_sref[b], slot)

    for slot in (0, 1):
        @pl.when((b < NB) & (first_sref[b] == 1) & (par_b == slot))
        def _sync(slot=slot):
            _wait(slot)

    spar_b = spar_sref[0]
    for slot in (0, 1):
        @pl.when((b == NB) & (spar_b == slot))
        def _sync_shared(slot=slot):
            _wait(slot)

    @pl.when(b >= NB)
    def _shared():
        tb = b - NB
        xs2 = xb_ref[pl.ds(tb * _TB, _TB), :]
        acc = jnp.zeros((_TB, D), dtype=jnp.float32)
        for c in range(H // _HB):
            sl = slice(c * _HB, (c + 1) * _HB)
            h = jax.nn.gelu(
                jnp.dot(xs2, wk_buf[spar_b, :, sl].astype(jnp.bfloat16),
                        preferred_element_type=jnp.float32)
                + bks_ref[0, sl])
            acc = acc + jnp.dot(h.astype(jnp.bfloat16),
                                wv_buf[spar_b, sl, :].astype(jnp.bfloat16),
                                preferred_element_type=jnp.float32)
        moe_ref[pl.ds(tb * _TB, _TB), :] += acc + bvs_ref[...]

    @pl.when((b < NB) & (valid_sref[b] == 1))
    def _compute():
        slot = (jax.lax.broadcasted_iota(jnp.int32, (_TB, T), 0)
                + b * _TB)
        g = ((p1_ref[...] == slot) | (p2_ref[...] == slot)
             ).astype(jnp.bfloat16)
        xs = jnp.dot(g, xb_ref[...],
                     preferred_element_type=jnp.float32).astype(jnp.bfloat16)
        acc = jnp.zeros((_TB, D), dtype=jnp.float32)
        for c in range(H // _HB):
            sl = slice(c * _HB, (c + 1) * _HB)
            h = jax.nn.gelu(
                jnp.dot(xs, wk_buf[par_b, :, sl].astype(jnp.bfloat16),
                        preferred_element_type=jnp.float32)
                + bk_ref[0, :, sl])
            acc = acc + jnp.dot(h.astype(jnp.bfloat16),
                                wv_buf[par_b, sl, :].astype(jnp.bfloat16),
                                preferred_element_type=jnp.float32)
        yb = (acc + bv_ref[0]).astype(jnp.bfloat16)
        slot_l = (jax.lax.broadcasted_iota(jnp.int32, (T, _TB), 1)
                  + b * _TB)
        cw = (jnp.where(p1c_ref[...] == slot_l, w1c_ref[...], 0.0)
              + jnp.where(p2c_ref[...] == slot_l, w2c_ref[...], 0.0))
        moe_ref[...] += jnp.dot(cw.astype(jnp.bfloat16), yb,
                                preferred_element_type=jnp.float32)


@functools.partial(jax.jit, static_argnames=())
def kernel(x, gate_kernel, Wk, bk, Wv, bv, Wk_s, bk_s, Wv_s, bv_s):
    B, S, D = x.shape
    T = B * S
    E = gate_kernel.shape[1]
    H = Wk.shape[2]
    NB = (2 * T) // _TB + E
    NS = NB * _TB
    NHB = H // _HB

    x2 = x.reshape(T, D)
    xb = x2.astype(jnp.bfloat16)
    bk3 = bk.reshape(E, 1, H)
    bv3 = bv.reshape(E, 1, D)
    bks2 = bk_s.reshape(1, H)
    bvs2 = bv_s.reshape(1, D)

    router = pl.pallas_call(
        functools.partial(_router_body, E=E, T=T, NB=NB),
        in_specs=[pl.BlockSpec((T, D), lambda: (0, 0)),
                  pl.BlockSpec((D, E), lambda: (0, 0))],
        out_specs=[pl.BlockSpec((T, 1), lambda: (0, 0)),
                   pl.BlockSpec((T, 1), lambda: (0, 0)),
                   pl.BlockSpec((T, 1), lambda: (0, 0)),
                   pl.BlockSpec((T, 1), lambda: (0, 0)),
                   pl.BlockSpec((64, 1), lambda: (0, 0)),
                   pl.BlockSpec((64, 1), lambda: (0, 0)),
                   pl.BlockSpec((64, 1), lambda: (0, 0)),
                   pl.BlockSpec((64, 1), lambda: (0, 0)),
                   pl.BlockSpec((64, 1), lambda: (0, 0)),
                   pl.BlockSpec((64, 1), lambda: (0, 0)),
                   pl.BlockSpec((64, 1), lambda: (0, 0))],
        out_shape=[jax.ShapeDtypeStruct((T, 1), jnp.int32),
                   jax.ShapeDtypeStruct((T, 1), jnp.int32),
                   jax.ShapeDtypeStruct((T, 1), jnp.float32),
                   jax.ShapeDtypeStruct((T, 1), jnp.float32),
                   jax.ShapeDtypeStruct((64, 1), jnp.int32),
                   jax.ShapeDtypeStruct((64, 1), jnp.int32),
                   jax.ShapeDtypeStruct((64, 1), jnp.int32),
                   jax.ShapeDtypeStruct((64, 1), jnp.int32),
                   jax.ShapeDtypeStruct((64, 1), jnp.int32),
                   jax.ShapeDtypeStruct((64, 1), jnp.int32),
                   jax.ShapeDtypeStruct((64, 1), jnp.int32)],
    )(x2, gate_kernel)
    (p1c, p2c, w1c, w2c, gid64, valid64,
     first64, issue64, ngid64, par64, spar64) = router
    p1r = p1c.reshape(1, T)
    p2r = p2c.reshape(1, T)
    NBG = NB + T // _TB
    gids = gid64.reshape(64)[:NBG]
    valid = valid64.reshape(64)[:NBG]
    first = first64.reshape(64)[:NBG]
    issue = issue64.reshape(64)[:NBG]
    ngid = ngid64.reshape(64)[:NBG]
    par = par64.reshape(64)[:NBG]
    spar = spar64.reshape(64)[:NBG]

    moe = pl.pallas_call(
        functools.partial(_expert_body, T=T, D=D, H=H, E=E, NB=NB),
        grid_spec=pltpu.PrefetchScalarGridSpec(
            num_scalar_prefetch=7,
            grid=(NBG,),
            in_specs=[
                pl.BlockSpec((T, D), lambda b, *s: (0, 0)),
                pl.BlockSpec((1, T), lambda b, *s: (0, 0)),
                pl.BlockSpec((1, T), lambda b, *s: (0, 0)),
                pl.BlockSpec((T, 1), lambda b, *s: (0, 0)),
                pl.BlockSpec((T, 1), lambda b, *s: (0, 0)),
                pl.BlockSpec((T, 1), lambda b, *s: (0, 0)),
                pl.BlockSpec((T, 1), lambda b, *s: (0, 0)),
                pl.BlockSpec(memory_space=pl.ANY),
                pl.BlockSpec((1, 1, H), lambda b, *s: (s[0][b], 0, 0)),
                pl.BlockSpec(memory_space=pl.ANY),
                pl.BlockSpec((1, 1, D), lambda b, *s: (s[0][b], 0, 0)),
                pl.BlockSpec(memory_space=pl.ANY),
                pl.BlockSpec(memory_space=pl.ANY),
                pl.BlockSpec((1, H), lambda b, *s: (0, 0)),
                pl.BlockSpec((1, D), lambda b, *s: (0, 0)),
            ],
            out_specs=pl.BlockSpec((T, D), lambda b, *s: (0, 0)),
            scratch_shapes=[
                pltpu.VMEM((2, D, H), jnp.float32),
                pltpu.VMEM((2, H, D), jnp.float32),
                pltpu.SemaphoreType.DMA,
                pltpu.SemaphoreType.DMA,
                pltpu.SemaphoreType.DMA,
                pltpu.SemaphoreType.DMA,
            ],
        ),
        out_shape=jax.ShapeDtypeStruct((T, D), jnp.float32),
        compiler_params=pltpu.CompilerParams(
            dimension_semantics=("arbitrary",)),
    )(gids, valid, first, issue, ngid, par, spar, xb, p1r, p2r,
      p1c, p2c, w1c, w2c, Wk, bk3, Wv, bv3, Wk_s, Wv_s, bks2, bvs2)

    return (moe.reshape(B, S, D), jnp.float32(0.0))
